# baseline (speedup 1.0000x reference)
import functools

import jax
import jax.numpy as jnp
from jax import lax
from jax.experimental import pallas as pl
from jax.experimental.pallas import tpu as pltpu
from jax.experimental.pallas import tpu_sc as plsc

B = 4096
F = 26
V = 100000
E = 16
A = 16

SEG = 12544
RC = SEG // 8
NW = 32
CHUNK = 128
NBUF = 4
PER_W = (B * F) // NW
CHUNKS_W = PER_W // CHUNK
BS = 8
RB = BS * F
GRID = B // BS


def _pack_body(t_ref, o_ref):
    for rc in range(8):
        parts = []
        for s in range(8):
            lo = s * SEG + rc * RC
            if lo + RC <= V:
                xs = t_ref[0, :, lo:lo + RC]
            else:
                nreal = max(V - lo, 0)
                tail = t_ref[0, :, lo:lo + nreal]
                xs = jnp.concatenate(
                    [tail, jnp.zeros((E, RC - nreal), jnp.float32)], axis=1)
            parts.append(xs.T)
        o_ref[rc * RC:(rc + 1) * RC, :] = jnp.concatenate(parts, axis=1)


def _pack(fm_second):
    fm2T = jnp.swapaxes(fm_second, 1, 2)
    return pl.pallas_call(
        _pack_body,
        grid=(F,),
        in_specs=[pl.BlockSpec((1, E, V), lambda f: (f, 0, 0))],
        out_specs=pl.BlockSpec((SEG, 128), lambda f: (f, 0)),
        out_shape=jax.ShapeDtypeStruct((F * SEG, 128), jnp.float32),
    )(fm2T)


def _sc_gather(p2_flat, fm1_wide, grow3d, g1div3d):
    mesh = plsc.VectorSubcoreMesh(core_axis_name="c", subcore_axis_name="s")

    @functools.partial(
        pl.kernel,
        mesh=mesh,
        out_type=[
            jax.ShapeDtypeStruct((B * F, 128), jnp.float32),
            jax.ShapeDtypeStruct((B * F, 16), jnp.float32),
        ],
        scratch_types=[
            pltpu.VMEM((CHUNKS_W, CHUNK), jnp.int32),
            pltpu.VMEM((CHUNKS_W, CHUNK), jnp.int32),
            pltpu.VMEM((NBUF, CHUNK, 128), jnp.float32),
            pltpu.VMEM((NBUF, CHUNK, 16), jnp.float32),
            pltpu.SemaphoreType.DMA,
            pltpu.SemaphoreType.DMA,
        ],
        compiler_params=pltpu.CompilerParams(use_tc_tiling_on_sc=False),
    )
    def k(p2_hbm, fm1_hbm, idx_hbm, div_hbm, e2_hbm, e1_hbm,
          idx_v, div_v, buf2_v, buf1_v, sem2, sem1):
        wid = lax.axis_index("s") * 2 + lax.axis_index("c")
        base = wid * PER_W
        pltpu.sync_copy(idx_hbm.at[wid], idx_v)
        pltpu.sync_copy(div_hbm.at[wid], div_v)

        def start(g):
            b = g % NBUF
            pltpu.async_copy(p2_hbm.at[idx_v.at[g]], buf2_v.at[b], sem2)
            pltpu.async_copy(fm1_hbm.at[div_v.at[g]], buf1_v.at[b], sem1)

        def wait(g):
            b = g % NBUF
            pltpu.make_async_copy(p2_hbm.at[idx_v.at[g]], buf2_v.at[b],
                                  sem2).wait()
            pltpu.make_async_copy(fm1_hbm.at[div_v.at[g]], buf1_v.at[b],
                                  sem1).wait()

        for g in range(NBUF):
            start(g)
        for g in range(CHUNKS_W):
            b = g % NBUF
            wait(g)
            pltpu.sync_copy(buf2_v.at[b],
                            e2_hbm.at[pl.ds(base + g * CHUNK, CHUNK)])
            pltpu.sync_copy(buf1_v.at[b],
                            e1_hbm.at[pl.ds(base + g * CHUNK, CHUNK)])
            if g + NBUF < CHUNKS_W:
                start(g + NBUF)

    return k(p2_flat, fm1_wide, grow3d, g1div3d)


def _tc_body(e2_ref, e1_ref, s8_ref, rem_ref, xv_ref, w_att_ref, h_ref,
             b_att_ref, p_ref, bias_ref, o_ref):
    xv = xv_ref[...]
    s8 = s8_ref[...]
    rows = e2_ref[...]
    x = jnp.zeros((RB, E), jnp.float32)
    for kk in range(8):
        x = x + jnp.where(s8 == kk, rows[:, 16 * kk:16 * (kk + 1)], 0.0)
    x = x * xv
    hv = h_ref[...]
    w = jnp.dot(hv, w_att_ref[...],
                preferred_element_type=jnp.float32)
    c = jnp.sum(b_att_ref[...] * hv)
    aw = jnp.concatenate([x * w, x * p_ref[...]], axis=0)
    g = lax.dot_general(aw, x, (((1,), (1,)), ((), ())),
                        preferred_element_type=jnp.float32)
    gw = g[:RB, :]
    gp = g[RB:, :]
    r = lax.broadcasted_iota(jnp.int32, (RB, RB), 0)
    col = lax.broadcasted_iota(jnp.int32, (RB, RB), 1)
    mask = ((r // F) == (col // F)) & ((r % F) < (col % F))
    eu = jnp.where(mask, jnp.exp(gw + c), 0.0)
    r1 = jnp.sum(eu, axis=1, keepdims=True)
    r2 = jnp.sum(eu * gp, axis=1, keepdims=True)
    rs = lax.broadcasted_iota(jnp.int32, (BS, RB), 0)
    cs = lax.broadcasted_iota(jnp.int32, (BS, RB), 1)
    sm = (rs == (cs // F)).astype(jnp.float32)
    s1 = jnp.dot(sm, r1, preferred_element_type=jnp.float32)
    s2 = jnp.dot(sm, r2, preferred_element_type=jnp.float32)
    lane = lax.broadcasted_iota(jnp.int32, (RB, 16), 1)
    e1col = jnp.sum(jnp.where(lane == rem_ref[...], e1_ref[...], 0.0),
                    axis=1, keepdims=True)
    e1s = jnp.dot(sm, e1col * xv,
                  preferred_element_type=jnp.float32)
    o_ref[...] = bias_ref[...] + e1s + s2 / s1


def kernel(Xi, Xv, fm_first, fm_second, bias, W_att, b_att, H, P):
    idx = Xi[:, :, 0].astype(jnp.int32)
    farr = jnp.arange(F, dtype=jnp.int32)[None, :]
    grow = farr * SEG + idx % SEG
    gs8 = (idx // SEG).reshape(B * F, 1)
    gidx = idx + farr * V
    grow3d = grow.reshape(NW, CHUNKS_W, CHUNK)
    g1div3d = (gidx // 16).reshape(NW, CHUNKS_W, CHUNK)
    g1rem = (gidx % 16).reshape(B * F, 1)

    p2 = _pack(fm_second)
    fm1_wide = fm_first.reshape((F * V) // 16, 16)

    e2g, e1g = _sc_gather(p2, fm1_wide, grow3d, g1div3d)

    out2d = pl.pallas_call(
        _tc_body,
        grid=(GRID,),
        in_specs=[
            pl.BlockSpec((RB, 128), lambda i: (i, 0)),
            pl.BlockSpec((RB, 16), lambda i: (i, 0)),
            pl.BlockSpec((RB, 1), lambda i: (i, 0)),
            pl.BlockSpec((RB, 1), lambda i: (i, 0)),
            pl.BlockSpec((RB, 1), lambda i: (i, 0)),
            pl.BlockSpec((A, E), lambda i: (0, 0)),
            pl.BlockSpec((1, A), lambda i: (0, 0)),
            pl.BlockSpec((1, A), lambda i: (0, 0)),
            pl.BlockSpec((1, E), lambda i: (0, 0)),
            pl.BlockSpec((1, 1), lambda i: (0, 0)),
        ],
        out_specs=pl.BlockSpec((BS, 1), lambda i: (i, 0)),
        out_shape=jax.ShapeDtypeStruct((B, 1), jnp.float32),
    )(e2g, e1g, gs8, g1rem, Xv.reshape(B * F, 1), W_att, H.reshape(1, A),
      b_att.reshape(1, A), P.reshape(1, E), bias.reshape(1, 1))
    return out2d.reshape(B)


# speedup vs baseline: 1.0810x; 1.0810x over previous
import functools

import jax
import jax.numpy as jnp
from jax import lax
from jax.experimental import pallas as pl
from jax.experimental.pallas import tpu as pltpu
from jax.experimental.pallas import tpu_sc as plsc

B = 4096
F = 26
V = 100000
E = 16
A = 16

SEG = 12544
RC = SEG // 8
NW = 32
CHUNK = 128
NBUF = 4
PER_W = (B * F) // NW
CHUNKS_W = PER_W // CHUNK
BS = 8
RB = BS * F
GRID = B // BS


def _pack_body(t_ref, o_ref):
    for rc in range(8):
        parts = []
        for s in range(8):
            lo = s * SEG + rc * RC
            if lo + RC <= V:
                xs = t_ref[0, :, lo:lo + RC]
            else:
                nreal = max(V - lo, 0)
                tail = t_ref[0, :, lo:lo + nreal]
                xs = jnp.concatenate(
                    [tail, jnp.zeros((E, RC - nreal), jnp.float32)], axis=1)
            parts.append(xs.T)
        o_ref[0, rc * RC:(rc + 1) * RC, :] = jnp.concatenate(parts, axis=1)


def _pack(fm_second):
    fm2T = jnp.swapaxes(fm_second, 1, 2)
    return pl.pallas_call(
        _pack_body,
        grid=(F,),
        in_specs=[pl.BlockSpec((1, E, V), lambda f: (f, 0, 0))],
        out_specs=pl.BlockSpec((1, SEG, 128), lambda f: (f, 0, 0)),
        out_shape=jax.ShapeDtypeStruct((F, SEG, 128), jnp.float32),
    )(fm2T)


def _sc_gather(p2_flat, fm1_wide, grow3d, g1div3d):
    mesh = plsc.VectorSubcoreMesh(core_axis_name="c", subcore_axis_name="s")

    @functools.partial(
        pl.kernel,
        mesh=mesh,
        out_type=[
            jax.ShapeDtypeStruct((B * F, 128), jnp.float32),
            jax.ShapeDtypeStruct((B * F, 16), jnp.float32),
        ],
        scratch_types=[
            pltpu.VMEM((CHUNKS_W, CHUNK), jnp.int32),
            pltpu.VMEM((CHUNKS_W, CHUNK), jnp.int32),
            pltpu.VMEM((NBUF, CHUNK, 128), jnp.float32),
            pltpu.VMEM((NBUF, CHUNK, 16), jnp.float32),
            pltpu.SemaphoreType.DMA,
            pltpu.SemaphoreType.DMA,
        ],
        compiler_params=pltpu.CompilerParams(use_tc_tiling_on_sc=False),
    )
    def k(p2_hbm, fm1_hbm, idx_hbm, div_hbm, e2_hbm, e1_hbm,
          idx_v, div_v, buf2_v, buf1_v, sem2, sem1):
        wid = lax.axis_index("s") * 2 + lax.axis_index("c")
        base = wid * PER_W
        pltpu.sync_copy(idx_hbm.at[wid], idx_v)
        pltpu.sync_copy(div_hbm.at[wid], div_v)

        def start(g):
            b = g % NBUF
            pltpu.async_copy(p2_hbm.at[idx_v.at[g]], buf2_v.at[b], sem2)
            pltpu.async_copy(fm1_hbm.at[div_v.at[g]], buf1_v.at[b], sem1)

        def wait(g):
            b = g % NBUF
            pltpu.make_async_copy(p2_hbm.at[idx_v.at[g]], buf2_v.at[b],
                                  sem2).wait()
            pltpu.make_async_copy(fm1_hbm.at[div_v.at[g]], buf1_v.at[b],
                                  sem1).wait()

        for g in range(NBUF):
            start(g)
        for g in range(CHUNKS_W):
            b = g % NBUF
            wait(g)
            pltpu.sync_copy(buf2_v.at[b],
                            e2_hbm.at[pl.ds(base + g * CHUNK, CHUNK)])
            pltpu.sync_copy(buf1_v.at[b],
                            e1_hbm.at[pl.ds(base + g * CHUNK, CHUNK)])
            if g + NBUF < CHUNKS_W:
                start(g + NBUF)

    return k(p2_flat, fm1_wide, grow3d, g1div3d)


def _tc_body(e2_ref, e1_ref, aux_ref, w_att_ref, h_ref,
             b_att_ref, p_ref, bias_ref, o_ref):
    xv = aux_ref[:, 0:1]
    s8 = aux_ref[:, 1:2]
    rem16 = aux_ref[:, 2:3]
    rows = e2_ref[...]
    x = jnp.zeros((RB, E), jnp.float32)
    for kk in range(8):
        x = x + jnp.where(s8 == float(kk), rows[:, 16 * kk:16 * (kk + 1)], 0.0)
    x = x * xv
    hv = h_ref[...]
    w = jnp.dot(hv, w_att_ref[...],
                preferred_element_type=jnp.float32)
    c = jnp.sum(b_att_ref[...] * hv)
    aw = jnp.concatenate([x * w, x * p_ref[...]], axis=0)
    g = lax.dot_general(aw, x, (((1,), (1,)), ((), ())),
                        preferred_element_type=jnp.float32)
    gw = g[:RB, :]
    gp = g[RB:, :]
    r = lax.broadcasted_iota(jnp.int32, (RB, RB), 0)
    col = lax.broadcasted_iota(jnp.int32, (RB, RB), 1)
    mask = ((r // F) == (col // F)) & ((r % F) < (col % F))
    eu = jnp.where(mask, jnp.exp(gw + c), 0.0)
    r1 = jnp.sum(eu, axis=1, keepdims=True)
    r2 = jnp.sum(eu * gp, axis=1, keepdims=True)
    rs = lax.broadcasted_iota(jnp.int32, (BS, RB), 0)
    cs = lax.broadcasted_iota(jnp.int32, (BS, RB), 1)
    sm = (rs == (cs // F)).astype(jnp.float32)
    s1 = jnp.dot(sm, r1, preferred_element_type=jnp.float32)
    s2 = jnp.dot(sm, r2, preferred_element_type=jnp.float32)
    lane = lax.broadcasted_iota(jnp.int32, (RB, 16), 1).astype(jnp.float32)
    e1col = jnp.sum(jnp.where(lane == rem16, e1_ref[...], 0.0),
                    axis=1, keepdims=True)
    e1s = jnp.dot(sm, e1col * xv,
                  preferred_element_type=jnp.float32)
    o_ref[...] = bias_ref[...] + e1s + s2 / s1


def kernel(Xi, Xv, fm_first, fm_second, bias, W_att, b_att, H, P):
    idx = Xi[:, :, 0].astype(jnp.int32)
    farr = jnp.arange(F, dtype=jnp.int32)[None, :]
    grow = farr * SEG + idx % SEG
    gidx = idx + farr * V
    grow3d = grow.reshape(NW, CHUNKS_W, CHUNK)
    g1div3d = (gidx // 16).reshape(NW, CHUNKS_W, CHUNK)
    aux = jnp.stack(
        [Xv.reshape(B * F),
         (idx // SEG).reshape(B * F).astype(jnp.float32),
         (gidx % 16).reshape(B * F).astype(jnp.float32),
         jnp.zeros((B * F,), jnp.float32)], axis=1)

    p2 = _pack(fm_second).reshape(F * SEG, 128)
    fm1_wide = fm_first.reshape((F * V) // 16, 16)

    e2g, e1g = _sc_gather(p2, fm1_wide, grow3d, g1div3d)

    out2d = pl.pallas_call(
        _tc_body,
        grid=(GRID,),
        in_specs=[
            pl.BlockSpec((RB, 128), lambda i: (i, 0)),
            pl.BlockSpec((RB, 16), lambda i: (i, 0)),
            pl.BlockSpec((RB, 4), lambda i: (i, 0)),
            pl.BlockSpec((A, E), lambda i: (0, 0)),
            pl.BlockSpec((1, A), lambda i: (0, 0)),
            pl.BlockSpec((1, A), lambda i: (0, 0)),
            pl.BlockSpec((1, E), lambda i: (0, 0)),
            pl.BlockSpec((1, 1), lambda i: (0, 0)),
        ],
        out_specs=pl.BlockSpec((BS, 1), lambda i: (i, 0)),
        out_shape=jax.ShapeDtypeStruct((B, 1), jnp.float32),
    )(e2g, e1g, aux, W_att, H.reshape(1, A),
      b_att.reshape(1, A), P.reshape(1, E), bias.reshape(1, 1))
    return out2d.reshape(B)
